# baseline (speedup 1.0000x reference)
import jax
import jax.numpy as jnp
from jax.experimental import pallas as pl
from jax.experimental.pallas import tpu as pltpu

DIM = 768
INTER = 256
E = 8
SI = 512
T = 2048


def _moe_kernel(x_ref, gw_ref, w13_ref, w2_ref, sw13_ref, sw2_ref, out_ref):
    xf = x_ref[...]
    xb = xf.astype(jnp.bfloat16)
    scores = jax.nn.sigmoid(
        jax.lax.dot_general(xf, gw_ref[...], (((1,), (1,)), ((), ())),
                            preferred_element_type=jnp.float32))
    m1 = jnp.max(scores, axis=1, keepdims=True)
    i1 = jnp.argmax(scores, axis=1)[:, None]
    eids = jax.lax.broadcasted_iota(jnp.int32, (T, E), 1)
    masked = jnp.where(eids == i1, -jnp.inf, scores)
    m2 = jnp.max(masked, axis=1, keepdims=True)
    i2 = jnp.argmax(masked, axis=1)[:, None]
    denom = m1 + m2
    combine = (jnp.where(eids == i1, m1 / denom, 0.0)
               + jnp.where(eids == i2, m2 / denom, 0.0))
    ab = jax.lax.dot(xb, sw13_ref[...], preferred_element_type=jnp.float32)
    hs = (jax.nn.silu(ab[:, :SI]) * ab[:, SI:]).astype(jnp.bfloat16)
    acc = jax.lax.dot(hs, sw2_ref[...], preferred_element_type=jnp.float32)
    for e in range(E):
        ce = combine[:, e:e + 1]
        abe = jax.lax.dot(xb, w13_ref[e],
                          preferred_element_type=jnp.float32)
        h = (jax.nn.silu(abe[:, :INTER]) * abe[:, INTER:] * ce
             ).astype(jnp.bfloat16)
        acc = acc + jax.lax.dot(h, w2_ref[e],
                                preferred_element_type=jnp.float32)
    out_ref[...] = acc


@jax.jit
def kernel(x, gate_w, w1, w2, w3, sw1, sw2, sw3):
    shape = x.shape
    xt = x.reshape(-1, DIM)
    w13 = jnp.concatenate([w1, w3], axis=2).astype(jnp.bfloat16)
    w2b = w2.astype(jnp.bfloat16)
    sw13 = jnp.concatenate([sw1, sw3], axis=1).astype(jnp.bfloat16)
    sw2b = sw2.astype(jnp.bfloat16)

    out = pl.pallas_call(
        _moe_kernel,
        out_shape=jax.ShapeDtypeStruct((T, DIM), jnp.float32),
    )(xt, gate_w, w13, w2b, sw13, sw2b)
    return out.reshape(shape)


# speedup vs baseline: 1.0451x; 1.0451x over previous
import jax
import jax.numpy as jnp
from jax.experimental import pallas as pl
from jax.experimental.pallas import tpu as pltpu

DIM = 768
INTER = 256
E = 8
SI = 512
T = 2048
TS = T // E


def _moe_kernel(x_ref, gw_ref, w13_ref, w2_ref, sw13_ref, sw2_ref,
                out_ref, combine_ref, xb_ref):
    e = pl.program_id(0)

    @pl.when(e == 0)
    def _init():
        xf = x_ref[...]
        xb_ref[...] = xf.astype(jnp.bfloat16)
        scores = jax.nn.sigmoid(
            jax.lax.dot_general(xf, gw_ref[...], (((1,), (1,)), ((), ())),
                                preferred_element_type=jnp.float32))
        m1 = jnp.max(scores, axis=1, keepdims=True)
        i1 = jnp.argmax(scores, axis=1)[:, None]
        eids = jax.lax.broadcasted_iota(jnp.int32, (T, E), 1)
        masked = jnp.where(eids == i1, -jnp.inf, scores)
        m2 = jnp.max(masked, axis=1, keepdims=True)
        i2 = jnp.argmax(masked, axis=1)[:, None]
        denom = m1 + m2
        combine_ref[...] = (jnp.where(eids == i1, m1 / denom, 0.0)
                            + jnp.where(eids == i2, m2 / denom, 0.0))
        xb = xb_ref[...]
        ab = jax.lax.dot(xb, sw13_ref[...], preferred_element_type=jnp.float32)
        hs = (jax.nn.silu(ab[:, :SI]) * ab[:, SI:]).astype(jnp.bfloat16)
        out_ref[...] = jax.lax.dot(hs, sw2_ref[...],
                                   preferred_element_type=jnp.float32)

    xb = xb_ref[...]
    cmb = combine_ref[...]
    lane = jax.lax.broadcasted_iota(jnp.int32, (T, E), 1)
    ce = jnp.sum(jnp.where(lane == e, cmb, 0.0), axis=1, keepdims=True)
    ab = jax.lax.dot(xb, w13_ref[0], preferred_element_type=jnp.float32)
    h = (jax.nn.silu(ab[:, :INTER]) * ab[:, INTER:] * ce).astype(jnp.bfloat16)
    out_ref[...] += jax.lax.dot(h, w2_ref[0], preferred_element_type=jnp.float32)


@jax.jit
def kernel(x, gate_w, w1, w2, w3, sw1, sw2, sw3):
    shape = x.shape
    xt = x.reshape(-1, DIM)
    w13 = jnp.concatenate([w1, w3], axis=2).astype(jnp.bfloat16)
    w2b = w2.astype(jnp.bfloat16)
    sw13 = jnp.concatenate([sw1, sw3], axis=1).astype(jnp.bfloat16)
    sw2b = sw2.astype(jnp.bfloat16)

    full = lambda shp: pl.BlockSpec(shp, lambda e: (0,) * len(shp))
    per_e = lambda shp: pl.BlockSpec((1,) + shp, lambda e: (e, 0, 0))

    out = pl.pallas_call(
        _moe_kernel,
        grid=(E,),
        in_specs=[
            full((T, DIM)),
            full((E, DIM)),
            per_e((DIM, 2 * INTER)),
            per_e((INTER, DIM)),
            full((DIM, 2 * SI)),
            full((SI, DIM)),
        ],
        out_specs=full((T, DIM)),
        out_shape=jax.ShapeDtypeStruct((T, DIM), jnp.float32),
        scratch_shapes=[
            pltpu.VMEM((T, E), jnp.float32),
            pltpu.VMEM((T, DIM), jnp.bfloat16),
        ],
    )(xt, gate_w, w13, w2b, sw13, sw2b)
    return out.reshape(shape)
